# Initial kernel scaffold; baseline (speedup 1.0000x reference)
import jax
import jax.numpy as jnp
from jax.experimental import pallas as pl


def kernel(x, edge_index, W0, b0, W1, b1, Wout, bout, Wm1, bm1, Wm2, bm2):
    raise NotImplementedError("write your pallas kernel here")


# speedup vs baseline: 19.3922x; 19.3922x over previous
import functools

import jax
import jax.numpy as jnp
from jax import lax
from jax.experimental import pallas as pl
from jax.experimental.pallas import tpu as pltpu
from jax.experimental.pallas import tpu_sc as plsc

N = 10000
E = 320000
D = 128

B = 80
EB = E // B
NW = 32
ROWS_W = EB // NW
NPAD = 10240
NT = 16
ROWS_T = NPAD // NT
RB = 1000
GRID = N // RB

_f32 = jnp.float32


def _deg_body(dst_hbm, ones_hbm, zeros1_hbm, out_hbm, idx_v, ones_v, deg_sh):
    cid = lax.axis_index("c")
    sid = lax.axis_index("s")
    wid = cid * NT + sid
    pltpu.sync_copy(zeros1_hbm, deg_sh.at[pl.ds(sid * 640, 640)])
    pltpu.sync_copy(dst_hbm.at[wid], idx_v)
    pltpu.sync_copy(ones_hbm, ones_v)
    plsc.subcore_barrier()

    @pl.loop(0, ROWS_W)
    def _(j):
        pltpu.sync_copy(ones_v, deg_sh.at[idx_v.at[j]], add=True)

    plsc.subcore_barrier()
    pltpu.sync_copy(deg_sh.at[pl.ds(sid * 640, 640)],
                    out_hbm.at[cid, 0, pl.ds(sid * 640, 640)])


def _scatter_body(h_hbm, src_hbm, dst_hbm, zeros2_hbm, out_hbm,
                  src_v, dst_v, rows_v, acc_sh):
    cid = lax.axis_index("c")
    sid = lax.axis_index("s")
    wid = cid * NT + sid
    pltpu.sync_copy(zeros2_hbm, acc_sh.at[pl.ds(sid * ROWS_T, ROWS_T)])
    pltpu.sync_copy(src_hbm.at[wid], src_v)
    pltpu.sync_copy(dst_hbm.at[wid], dst_v)
    plsc.subcore_barrier()

    @pl.loop(0, ROWS_W)
    def _(j):
        pltpu.sync_copy(h_hbm.at[src_v.at[j]], rows_v)
        pltpu.sync_copy(rows_v, acc_sh.at[dst_v.at[j]], add=True)

    plsc.subcore_barrier()
    pltpu.sync_copy(acc_sh.at[pl.ds(sid * ROWS_T, ROWS_T)],
                    out_hbm.at[cid, pl.ds(sid * ROWS_T, ROWS_T)])


@functools.lru_cache(maxsize=None)
def _sc_kernels():
    mesh = plsc.VectorSubcoreMesh(core_axis_name="c", subcore_axis_name="s",
                                  num_cores=2, num_subcores=NT)
    deg = pl.kernel(
        _deg_body,
        out_type=jax.ShapeDtypeStruct((2, 1, NPAD), _f32),
        mesh=mesh,
        scratch_types=[
            pltpu.VMEM((ROWS_W, B), jnp.int32),
            pltpu.VMEM((B,), _f32),
            pltpu.VMEM_SHARED((NPAD,), _f32),
        ],
    )
    scat = pl.kernel(
        _scatter_body,
        out_type=jax.ShapeDtypeStruct((2, NPAD, D), _f32),
        mesh=mesh,
        scratch_types=[
            pltpu.VMEM((ROWS_W, B), jnp.int32),
            pltpu.VMEM((ROWS_W, B), jnp.int32),
            pltpu.VMEM((B, D), _f32),
            pltpu.VMEM_SHARED((NPAD, D), _f32),
        ],
    )
    return deg, scat


def _tc1_body(x_ref, w_ref, d0_ref, d1_ref, hs_ref, dinv_ref):
    deg = d0_ref[...] + d1_ref[...] + 1.0
    dinv = lax.rsqrt(deg)
    dinv_ref[...] = dinv
    h = jnp.dot(x_ref[...], w_ref[...], preferred_element_type=_f32)
    hs_ref[...] = h * dinv


_tc1 = pl.pallas_call(
    _tc1_body,
    grid=(GRID,),
    in_specs=[
        pl.BlockSpec((RB, D), lambda i: (i, 0)),
        pl.BlockSpec((D, D), lambda i: (0, 0)),
        pl.BlockSpec((RB, 1), lambda i: (i, 0)),
        pl.BlockSpec((RB, 1), lambda i: (i, 0)),
    ],
    out_specs=[
        pl.BlockSpec((RB, D), lambda i: (i, 0)),
        pl.BlockSpec((RB, 1), lambda i: (i, 0)),
    ],
    out_shape=[
        jax.ShapeDtypeStruct((N, D), _f32),
        jax.ShapeDtypeStruct((N, 1), _f32),
    ],
)


def _tc2_body(acc_ref, hs_ref, dinv_ref, b_ref, w_ref, out_ref):
    dinv = dinv_ref[...]
    pre = dinv * (acc_ref[0] + acc_ref[1] + hs_ref[...]) + b_ref[...]
    h = jnp.where(pre >= 0, pre, 0.01 * pre)
    out_ref[...] = jnp.dot(h, w_ref[...], preferred_element_type=_f32) * dinv


_tc2 = pl.pallas_call(
    _tc2_body,
    grid=(GRID,),
    in_specs=[
        pl.BlockSpec((2, RB, D), lambda i: (0, i, 0)),
        pl.BlockSpec((RB, D), lambda i: (i, 0)),
        pl.BlockSpec((RB, 1), lambda i: (i, 0)),
        pl.BlockSpec((1, D), lambda i: (0, 0)),
        pl.BlockSpec((D, D), lambda i: (0, 0)),
    ],
    out_specs=pl.BlockSpec((RB, D), lambda i: (i, 0)),
    out_shape=jax.ShapeDtypeStruct((N, D), _f32),
)


def _tc3_body(acc_ref, hs_ref, dinv_ref, b_ref, wo_ref, bo_ref,
              wm1_ref, bm1_ref, wm2_ref, bm2_ref, y_ref, vn_ref, sum_ref):
    i = pl.program_id(0)
    pre = dinv_ref[...] * (acc_ref[0] + acc_ref[1] + hs_ref[...]) + b_ref[...]
    h2 = jnp.where(pre >= 0, pre, 0.01 * pre)
    y_ref[...] = jnp.dot(h2, wo_ref[...], preferred_element_type=_f32) + bo_ref[...]
    part = jnp.sum(h2, axis=0, keepdims=True)

    @pl.when(i == 0)
    def _():
        sum_ref[...] = part
        vn_ref[...] = jnp.zeros((1, D), _f32)

    @pl.when(i > 0)
    def _():
        sum_ref[...] += part

    @pl.when(i == GRID - 1)
    def _():
        v = jnp.dot(sum_ref[...], wm1_ref[...], preferred_element_type=_f32)
        v = jnp.maximum(v + bm1_ref[...], 0.0)
        v = jnp.dot(v, wm2_ref[...], preferred_element_type=_f32)
        vn_ref[...] = jnp.maximum(v + bm2_ref[...], 0.0)


_tc3 = pl.pallas_call(
    _tc3_body,
    grid=(GRID,),
    in_specs=[
        pl.BlockSpec((2, RB, D), lambda i: (0, i, 0)),
        pl.BlockSpec((RB, D), lambda i: (i, 0)),
        pl.BlockSpec((RB, 1), lambda i: (i, 0)),
        pl.BlockSpec((1, D), lambda i: (0, 0)),
        pl.BlockSpec((D, D), lambda i: (0, 0)),
        pl.BlockSpec((1, D), lambda i: (0, 0)),
        pl.BlockSpec((D, D), lambda i: (0, 0)),
        pl.BlockSpec((1, D), lambda i: (0, 0)),
        pl.BlockSpec((D, D), lambda i: (0, 0)),
        pl.BlockSpec((1, D), lambda i: (0, 0)),
    ],
    out_specs=[
        pl.BlockSpec((RB, D), lambda i: (i, 0)),
        pl.BlockSpec((1, D), lambda i: (0, 0)),
    ],
    out_shape=[
        jax.ShapeDtypeStruct((N, D), _f32),
        jax.ShapeDtypeStruct((1, D), _f32),
    ],
    scratch_shapes=[pltpu.VMEM((1, D), _f32)],
    compiler_params=pltpu.CompilerParams(
        dimension_semantics=("arbitrary",)),
)


@jax.jit
def kernel(x, edge_index, W0, b0, W1, b1, Wout, bout, Wm1, bm1, Wm2, bm2):
    src3d = edge_index[0].reshape(NW, ROWS_W, B)
    dst3d = edge_index[1].reshape(NW, ROWS_W, B)
    ones_b = jnp.ones((B,), _f32)
    zeros1 = jnp.zeros((640,), _f32)
    zeros2 = jnp.zeros((ROWS_T, D), _f32)

    deg_sc, scatter_sc = _sc_kernels()
    degp = deg_sc(dst3d, ones_b, zeros1)
    d0 = degp[0, 0, :N].reshape(N, 1)
    d1 = degp[1, 0, :N].reshape(N, 1)

    h0s, dinv = _tc1(x, W0, d0, d1)
    acc0 = scatter_sc(h0s, src3d, dst3d, zeros2)
    h1s = _tc2(acc0, h0s, dinv, b0.reshape(1, D), W1)
    acc1 = scatter_sc(h1s, src3d, dst3d, zeros2)
    y, vn = _tc3(acc1, h1s, dinv, b1.reshape(1, D),
                 Wout, bout.reshape(1, D), Wm1, bm1.reshape(1, D),
                 Wm2, bm2.reshape(1, D))
    return y, vn.reshape(D)


# speedup vs baseline: 32.1872x; 1.6598x over previous
import functools

import jax
import jax.numpy as jnp
from jax import lax
from jax.experimental import pallas as pl
from jax.experimental.pallas import tpu as pltpu
from jax.experimental.pallas import tpu_sc as plsc

N = 10000
E = 320000
D = 128

NW = 32
BD = 80
ROWS_WD = E // BD // NW
B = 125
ROWS_W = E // B // NW
NBUF = 2
NIDX = 4
NGRPI = ROWS_W // NIDX
NPAD = 10240
NT = 16
ROWS_T = NPAD // NT
RB = 1000
GRID = N // RB

_f32 = jnp.float32


def _deg_body(dst_hbm, ones_hbm, zeros1_hbm, out_hbm, idx_v, ones_v, deg_sh):
    cid = lax.axis_index("c")
    sid = lax.axis_index("s")
    wid = cid * NT + sid
    pltpu.sync_copy(zeros1_hbm, deg_sh.at[pl.ds(sid * 640, 640)])
    pltpu.sync_copy(dst_hbm.at[wid], idx_v)
    pltpu.sync_copy(ones_hbm, ones_v)
    plsc.subcore_barrier()

    @pl.loop(0, ROWS_WD)
    def _(j):
        pltpu.sync_copy(ones_v, deg_sh.at[idx_v.at[j]], add=True)

    plsc.subcore_barrier()
    pltpu.sync_copy(deg_sh.at[pl.ds(sid * 640, 640)],
                    out_hbm.at[cid, 0, pl.ds(sid * 640, 640)])


def _scatter_body(h_hbm, ei_hbm, zeros2_hbm, out_hbm,
                  idx_v, rows_v, acc_sh, isem, gsem):
    cid = lax.axis_index("c")
    sid = lax.axis_index("s")
    wid = cid * NT + sid
    pltpu.sync_copy(zeros2_hbm, acc_sh.at[pl.ds(sid * ROWS_T, ROWS_T)])
    plsc.subcore_barrier()

    for k in range(NIDX):
        pltpu.async_copy(ei_hbm.at[wid, k], idx_v.at[k], isem.at[k])
    for b in range(NBUF):
        pltpu.make_async_copy(ei_hbm.at[wid, b], idx_v.at[b],
                              isem.at[b]).wait()
        pltpu.async_copy(h_hbm.at[idx_v.at[b, 0]], rows_v.at[b], gsem.at[b])

    @pl.loop(0, NGRPI - 1)
    def _(g):
        j0 = g * NIDX
        for k in range(NIDX):
            b = k % NBUF
            ks = (k + NBUF) % NIDX
            pltpu.make_async_copy(h_hbm.at[idx_v.at[k, 0]], rows_v.at[b],
                                  gsem.at[b]).wait()
            pltpu.sync_copy(rows_v.at[b], acc_sh.at[idx_v.at[k, 1]], add=True)
            pltpu.async_copy(ei_hbm.at[wid, j0 + k + NIDX], idx_v.at[k],
                             isem.at[k])
            pltpu.make_async_copy(ei_hbm.at[wid, k], idx_v.at[ks],
                                  isem.at[ks]).wait()
            pltpu.async_copy(h_hbm.at[idx_v.at[ks, 0]], rows_v.at[b],
                             gsem.at[b])

    for k in range(NIDX):
        b = k % NBUF
        ks = (k + NBUF) % NIDX
        pltpu.make_async_copy(h_hbm.at[idx_v.at[k, 0]], rows_v.at[b],
                              gsem.at[b]).wait()
        pltpu.sync_copy(rows_v.at[b], acc_sh.at[idx_v.at[k, 1]], add=True)
        if k < NIDX - NBUF:
            pltpu.make_async_copy(ei_hbm.at[wid, k], idx_v.at[ks],
                                  isem.at[ks]).wait()
            pltpu.async_copy(h_hbm.at[idx_v.at[ks, 0]], rows_v.at[b],
                             gsem.at[b])

    plsc.subcore_barrier()
    pltpu.sync_copy(acc_sh.at[pl.ds(sid * ROWS_T, ROWS_T)],
                    out_hbm.at[cid, pl.ds(sid * ROWS_T, ROWS_T)])


@functools.lru_cache(maxsize=None)
def _sc_kernels():
    mesh = plsc.VectorSubcoreMesh(core_axis_name="c", subcore_axis_name="s",
                                  num_cores=2, num_subcores=NT)
    deg = pl.kernel(
        _deg_body,
        out_type=jax.ShapeDtypeStruct((2, 1, NPAD), _f32),
        mesh=mesh,
        scratch_types=[
            pltpu.VMEM((ROWS_WD, BD), jnp.int32),
            pltpu.VMEM((BD,), _f32),
            pltpu.VMEM_SHARED((NPAD,), _f32),
        ],
    )
    scat = pl.kernel(
        _scatter_body,
        out_type=jax.ShapeDtypeStruct((2, NPAD, D), _f32),
        mesh=mesh,
        scratch_types=[
            pltpu.VMEM((NIDX, 2, B), jnp.int32),
            pltpu.VMEM((NBUF, B, D), _f32),
            pltpu.VMEM_SHARED((NPAD, D), _f32),
            pltpu.SemaphoreType.DMA((NIDX,)),
            pltpu.SemaphoreType.DMA((NBUF,)),
        ],
    )
    return deg, scat


def _tc1_body(x_ref, w_ref, d0_ref, d1_ref, hs_ref, dinv_ref):
    deg = d0_ref[...] + d1_ref[...] + 1.0
    dinv = lax.rsqrt(deg)
    dinv_ref[...] = dinv
    h = jnp.dot(x_ref[...], w_ref[...], preferred_element_type=_f32)
    hs_ref[...] = h * dinv


_tc1 = pl.pallas_call(
    _tc1_body,
    grid=(GRID,),
    in_specs=[
        pl.BlockSpec((RB, D), lambda i: (i, 0)),
        pl.BlockSpec((D, D), lambda i: (0, 0)),
        pl.BlockSpec((RB, 1), lambda i: (i, 0)),
        pl.BlockSpec((RB, 1), lambda i: (i, 0)),
    ],
    out_specs=[
        pl.BlockSpec((RB, D), lambda i: (i, 0)),
        pl.BlockSpec((RB, 1), lambda i: (i, 0)),
    ],
    out_shape=[
        jax.ShapeDtypeStruct((N, D), _f32),
        jax.ShapeDtypeStruct((N, 1), _f32),
    ],
)


def _tc2_body(acc_ref, hs_ref, dinv_ref, b_ref, w_ref, out_ref):
    dinv = dinv_ref[...]
    pre = dinv * (acc_ref[0] + acc_ref[1] + hs_ref[...]) + b_ref[...]
    h = jnp.where(pre >= 0, pre, 0.01 * pre)
    out_ref[...] = jnp.dot(h, w_ref[...], preferred_element_type=_f32) * dinv


_tc2 = pl.pallas_call(
    _tc2_body,
    grid=(GRID,),
    in_specs=[
        pl.BlockSpec((2, RB, D), lambda i: (0, i, 0)),
        pl.BlockSpec((RB, D), lambda i: (i, 0)),
        pl.BlockSpec((RB, 1), lambda i: (i, 0)),
        pl.BlockSpec((1, D), lambda i: (0, 0)),
        pl.BlockSpec((D, D), lambda i: (0, 0)),
    ],
    out_specs=pl.BlockSpec((RB, D), lambda i: (i, 0)),
    out_shape=jax.ShapeDtypeStruct((N, D), _f32),
)


def _tc3_body(acc_ref, hs_ref, dinv_ref, b_ref, wo_ref, bo_ref,
              wm1_ref, bm1_ref, wm2_ref, bm2_ref, y_ref, vn_ref, sum_ref):
    i = pl.program_id(0)
    pre = dinv_ref[...] * (acc_ref[0] + acc_ref[1] + hs_ref[...]) + b_ref[...]
    h2 = jnp.where(pre >= 0, pre, 0.01 * pre)
    y_ref[...] = jnp.dot(h2, wo_ref[...], preferred_element_type=_f32) + bo_ref[...]
    part = jnp.sum(h2, axis=0, keepdims=True)

    @pl.when(i == 0)
    def _():
        sum_ref[...] = part
        vn_ref[...] = jnp.zeros((1, D), _f32)

    @pl.when(i > 0)
    def _():
        sum_ref[...] += part

    @pl.when(i == GRID - 1)
    def _():
        v = jnp.dot(sum_ref[...], wm1_ref[...], preferred_element_type=_f32)
        v = jnp.maximum(v + bm1_ref[...], 0.0)
        v = jnp.dot(v, wm2_ref[...], preferred_element_type=_f32)
        vn_ref[...] = jnp.maximum(v + bm2_ref[...], 0.0)


_tc3 = pl.pallas_call(
    _tc3_body,
    grid=(GRID,),
    in_specs=[
        pl.BlockSpec((2, RB, D), lambda i: (0, i, 0)),
        pl.BlockSpec((RB, D), lambda i: (i, 0)),
        pl.BlockSpec((RB, 1), lambda i: (i, 0)),
        pl.BlockSpec((1, D), lambda i: (0, 0)),
        pl.BlockSpec((D, D), lambda i: (0, 0)),
        pl.BlockSpec((1, D), lambda i: (0, 0)),
        pl.BlockSpec((D, D), lambda i: (0, 0)),
        pl.BlockSpec((1, D), lambda i: (0, 0)),
        pl.BlockSpec((D, D), lambda i: (0, 0)),
        pl.BlockSpec((1, D), lambda i: (0, 0)),
    ],
    out_specs=[
        pl.BlockSpec((RB, D), lambda i: (i, 0)),
        pl.BlockSpec((1, D), lambda i: (0, 0)),
    ],
    out_shape=[
        jax.ShapeDtypeStruct((N, D), _f32),
        jax.ShapeDtypeStruct((1, D), _f32),
    ],
    scratch_shapes=[pltpu.VMEM((1, D), _f32)],
    compiler_params=pltpu.CompilerParams(
        dimension_semantics=("arbitrary",)),
)


@jax.jit
def kernel(x, edge_index, W0, b0, W1, b1, Wout, bout, Wm1, bm1, Wm2, bm2):
    ei3d = jnp.stack([edge_index[0].reshape(NW, ROWS_W, B),
                      edge_index[1].reshape(NW, ROWS_W, B)], axis=2)
    dst3d_deg = edge_index[1].reshape(NW, ROWS_WD, BD)
    ones_b = jnp.ones((BD,), _f32)
    zeros1 = jnp.zeros((640,), _f32)
    zeros2 = jnp.zeros((ROWS_T, D), _f32)

    deg_sc, scatter_sc = _sc_kernels()
    degp = deg_sc(dst3d_deg, ones_b, zeros1)
    d0 = degp[0, 0, :N].reshape(N, 1)
    d1 = degp[1, 0, :N].reshape(N, 1)

    h0s, dinv = _tc1(x, W0, d0, d1)
    acc0 = scatter_sc(h0s, ei3d, zeros2)
    h1s = _tc2(acc0, h0s, dinv, b0.reshape(1, D), W1)
    acc1 = scatter_sc(h1s, ei3d, zeros2)
    y, vn = _tc3(acc1, h1s, dinv, b1.reshape(1, D),
                 Wout, bout.reshape(1, D), Wm1, bm1.reshape(1, D),
                 Wm2, bm2.reshape(1, D))
    return y, vn.reshape(D)
